# baseline (speedup 1.0000x reference)
import functools

import jax
import jax.numpy as jnp
from jax import lax
from jax.experimental import pallas as pl
from jax.experimental.pallas import tpu as pltpu
from jax.experimental.pallas import tpu_sc as plsc

N_TOKENS = 16 * 1024
DIM = 256
K_CODES = 8192
WIN = 2736

N_TILE = 256


def _bf16_rne(v):
    u = lax.bitcast_convert_type(v, jnp.uint32)
    u = (u + jnp.uint32(0x7FFF) + ((u >> 16) & jnp.uint32(1))) & jnp.uint32(0xFFFF0000)
    return lax.bitcast_convert_type(u, jnp.float32)


def _argmin_body(x_ref, e_ref, xx_ref, ee_ref, out_ref):
    xb = x_ref[...]
    eb = e_ref[...]
    mm = lax.dot_general(xb.astype(jnp.bfloat16), eb.astype(jnp.bfloat16),
                         (((1,), (1,)), ((), ())),
                         preferred_element_type=jnp.float32)
    xx = xx_ref[0, 0, :][:, None]
    ee = ee_ref[0, 0, :][None, :]
    dist = -((xx - 2.0 * mm) + ee)
    kidx = lax.broadcasted_iota(jnp.int32, dist.shape, 1)
    acc_v = jnp.full((N_TILE,), -jnp.inf, jnp.float32)
    acc_i = jnp.zeros((N_TILE,), jnp.int32)
    for w in range(3):
        lo, hi = w * WIN, min((w + 1) * WIN, K_CODES)
        inwin = (kidx >= lo) & (kidx < hi)
        vals = jnp.where(inwin, dist, -jnp.inf)
        m = jnp.max(vals, axis=1)
        idx = jnp.min(jnp.where(vals == m[:, None], kidx, K_CODES), axis=1)
        upd = m > acc_v
        acc_i = jnp.where(upd, idx, acc_i)
        acc_v = jnp.where(upd, _bf16_rne(m), acc_v)
    out_ref[0, 0, :] = acc_i


def _nearest_code(xf, embed, xx3, ee3):
    nblk = N_TOKENS // N_TILE
    ind3 = pl.pallas_call(
        _argmin_body,
        grid=(nblk,),
        in_specs=[
            pl.BlockSpec((N_TILE, DIM), lambda i: (i, 0)),
            pl.BlockSpec((K_CODES, DIM), lambda i: (0, 0)),
            pl.BlockSpec((1, 1, N_TILE), lambda i: (i, 0, 0)),
            pl.BlockSpec((1, 1, K_CODES), lambda i: (0, 0, 0)),
        ],
        out_specs=pl.BlockSpec((1, 1, N_TILE), lambda i: (i, 0, 0)),
        out_shape=jax.ShapeDtypeStruct((nblk, 1, N_TILE), jnp.int32),
    )(xf, embed, xx3, ee3)
    return ind3.reshape(N_TOKENS)


@functools.cache
def _make_gather():
    info = plsc.get_sparse_core_info()
    nw = info.num_cores * info.num_subcores
    b_per_w = N_TOKENS // nw
    chunk = 128
    n_chunks = b_per_w // chunk
    mesh = plsc.VectorSubcoreMesh(core_axis_name="c", subcore_axis_name="s")

    @functools.partial(
        pl.kernel, mesh=mesh,
        out_type=jax.ShapeDtypeStruct((N_TOKENS, DIM), jnp.float32),
        scratch_types=[
            pltpu.VMEM((chunk,), jnp.int32),
            pltpu.VMEM((chunk, DIM), jnp.float32),
            pltpu.SemaphoreType.DMA,
        ],
    )
    def gather(table_hbm, idx_hbm, out_hbm, idx_v, rows_v, sem):
        wid = lax.axis_index("s") * info.num_cores + lax.axis_index("c")
        base = wid * b_per_w
        for j in range(n_chunks):
            off = base + j * chunk
            pltpu.sync_copy(idx_hbm.at[pl.ds(off, chunk)], idx_v)
            pltpu.async_copy(table_hbm.at[idx_v], rows_v, sem).wait()
            pltpu.sync_copy(rows_v, out_hbm.at[pl.ds(off, chunk)])

    return gather


def kernel(x, embed):
    shape = x.shape
    xf = x.reshape(-1, shape[-1])
    et = embed.T
    xx = jnp.sum(xf ** 2, axis=1, keepdims=True)
    ee = jnp.sum(et ** 2, axis=0, keepdims=True)
    xx3 = xx.reshape(N_TOKENS // N_TILE, 1, N_TILE)
    ee3 = ee.reshape(1, 1, K_CODES)
    ind = _nearest_code(xf, embed, xx3, ee3)
    quantize = _make_gather()(embed, ind)
    return (quantize.reshape(shape), ind.reshape(shape[:-1]))


# speedup vs baseline: 1.6734x; 1.6734x over previous
import functools

import jax
import jax.numpy as jnp
from jax import lax
from jax.experimental import pallas as pl
from jax.experimental.pallas import tpu as pltpu
from jax.experimental.pallas import tpu_sc as plsc

N_TOKENS = 16 * 1024
DIM = 256
K_CODES = 8192
WIN = 2736

N_TILE = 256


def _bf16_rne(v):
    u = lax.bitcast_convert_type(v, jnp.uint32)
    u = (u + jnp.uint32(0x7FFF) + ((u >> 16) & jnp.uint32(1))) & jnp.uint32(0xFFFF0000)
    return lax.bitcast_convert_type(u, jnp.float32)


def _argmin_body(x_ref, e_ref, xx_ref, ee_ref, out_ref):
    xb = x_ref[...]
    eb = e_ref[...]
    xb2 = xb + xb
    mm2 = lax.dot_general(eb.astype(jnp.bfloat16), xb2.astype(jnp.bfloat16),
                          (((1,), (1,)), ((), ())),
                          preferred_element_type=jnp.float32)
    xx = xx_ref[0, 0, :][None, :]
    ee = ee_ref[0, 0, :][:, None]
    a = (xx - mm2) + ee
    acc_v = jnp.full((N_TILE,), jnp.inf, jnp.float32)
    acc_i = jnp.zeros((N_TILE,), jnp.int32)
    for w in range(3):
        lo, hi = w * WIN, min((w + 1) * WIN, K_CODES)
        s = a[lo:hi, :]
        m = jnp.min(s, axis=0)
        ki = lax.broadcasted_iota(jnp.int32, s.shape, 0) + lo
        idx = jnp.min(jnp.where(s == m[None, :], ki, K_CODES), axis=0)
        upd = m < acc_v
        acc_i = jnp.where(upd, idx, acc_i)
        acc_v = jnp.where(upd, _bf16_rne(m), acc_v)
    out_ref[0, 0, :] = acc_i


def _nearest_code(xf, embed, xx3, ee3):
    nblk = N_TOKENS // N_TILE
    ind3 = pl.pallas_call(
        _argmin_body,
        grid=(nblk,),
        in_specs=[
            pl.BlockSpec((N_TILE, DIM), lambda i: (i, 0)),
            pl.BlockSpec((K_CODES, DIM), lambda i: (0, 0)),
            pl.BlockSpec((1, 1, N_TILE), lambda i: (i, 0, 0)),
            pl.BlockSpec((1, 1, K_CODES), lambda i: (0, 0, 0)),
        ],
        out_specs=pl.BlockSpec((1, 1, N_TILE), lambda i: (i, 0, 0)),
        out_shape=jax.ShapeDtypeStruct((nblk, 1, N_TILE), jnp.int32),
    )(xf, embed, xx3, ee3)
    return ind3.reshape(N_TOKENS)


@functools.cache
def _make_gather():
    info = plsc.get_sparse_core_info()
    nw = info.num_cores * info.num_subcores
    b_per_w = N_TOKENS // nw
    chunk = 128
    n_chunks = b_per_w // chunk
    mesh = plsc.VectorSubcoreMesh(core_axis_name="c", subcore_axis_name="s")

    @functools.partial(
        pl.kernel, mesh=mesh,
        out_type=jax.ShapeDtypeStruct((N_TOKENS, DIM), jnp.float32),
        scratch_types=[
            pltpu.VMEM((chunk,), jnp.int32),
            pltpu.VMEM((chunk, DIM), jnp.float32),
            pltpu.SemaphoreType.DMA,
        ],
    )
    def gather(table_hbm, idx_hbm, out_hbm, idx_v, rows_v, sem):
        wid = lax.axis_index("s") * info.num_cores + lax.axis_index("c")
        base = wid * b_per_w
        for j in range(n_chunks):
            off = base + j * chunk
            pltpu.sync_copy(idx_hbm.at[pl.ds(off, chunk)], idx_v)
            pltpu.async_copy(table_hbm.at[idx_v], rows_v, sem).wait()
            pltpu.sync_copy(rows_v, out_hbm.at[pl.ds(off, chunk)])

    return gather


def kernel(x, embed):
    shape = x.shape
    xf = x.reshape(-1, shape[-1])
    et = embed.T
    xx = jnp.sum(xf ** 2, axis=1, keepdims=True)
    ee = jnp.sum(et ** 2, axis=0, keepdims=True)
    xx3 = xx.reshape(N_TOKENS // N_TILE, 1, N_TILE)
    ee3 = ee.reshape(1, 1, K_CODES)
    ind = _nearest_code(xf, embed, xx3, ee3)
    quantize = _make_gather()(embed, ind)
    return (quantize.reshape(shape), ind.reshape(shape[:-1]))
